# Initial kernel scaffold; baseline (speedup 1.0000x reference)
import jax
import jax.numpy as jnp
from jax.experimental import pallas as pl


def kernel(z, z_embed, embeds):
    raise NotImplementedError("write your pallas kernel here")


# speedup vs baseline: 6.4642x; 6.4642x over previous
import functools

import jax
import jax.numpy as jnp
import numpy as np
from jax import lax
from jax.experimental import pallas as pl
from jax.experimental.pallas import tpu as pltpu
from jax.experimental.pallas import tpu_sc as plsc

MAX_Z = 36
N_ORB = 13
EMBED_DIM = 64
SUB_CAPS = [2, 2, 3, 3, 2, 3, 3, 2, 4, 3, 3, 3, 3]

NC, NS = 2, 16
NW = NC * NS
BLK = 40
ROW = N_ORB * EMBED_DIM


def _elec_idx_const() -> np.ndarray:
    t = np.zeros((MAX_Z + 1, N_ORB), dtype=np.int32)
    for zz in range(1, MAX_Z + 1):
        rem = zz
        for col, cap in enumerate(SUB_CAPS):
            e = min(rem, cap)
            t[zz, col] = e
            rem -= e
            if rem == 0:
                break
    idx = (np.arange(N_ORB, dtype=np.int32)[None, :] * 5 + t).reshape(-1)
    pad = np.zeros(512, dtype=np.int32)
    pad[: idx.size] = idx
    return pad.reshape(4, 128)


_MESH = plsc.VectorSubcoreMesh(core_axis_name="c", subcore_axis_name="s")
_NROWS = (MAX_Z + 1) * N_ORB


def _fuse_body(ef_hbm, idx_hbm, fused_hbm, idx_v, fused_v, sem):
    wid = lax.axis_index("s") * NC + lax.axis_index("c")

    @pl.when(wid == 0)
    def _():
        pltpu.sync_copy(idx_hbm, idx_v)
        for k in range(4):
            pltpu.async_copy(
                ef_hbm.at[idx_v.at[k]], fused_v.at[pl.ds(k * 128, 128)], sem
            ).wait()
        pltpu.sync_copy(fused_v.at[pl.ds(0, _NROWS)], fused_hbm)


_SC_PARAMS = pltpu.CompilerParams(use_tc_tiling_on_sc=False)

_fuse = pl.kernel(
    _fuse_body,
    out_type=jax.ShapeDtypeStruct((_NROWS, EMBED_DIM), jnp.float32),
    mesh=_MESH,
    compiler_params=_SC_PARAMS,
    scratch_types=[
        pltpu.VMEM((4, 128), jnp.int32),
        pltpu.VMEM((512, EMBED_DIM), jnp.float32),
        pltpu.SemaphoreType.DMA,
    ],
)


def _combine_body(n_node, z_hbm, ze_hbm, fused_hbm, out_hbm,
                  z_v, ze_v, rows_v, sem, sem2):
    wid = lax.axis_index("s") * NC + lax.axis_index("c")
    n_blocks = n_node // BLK
    max_iter = (n_blocks + NW - 1) // NW

    def blk_body(i, carry):
        blk = i * NW + wid

        @pl.when(blk < n_blocks)
        def _():
            start = blk * BLK
            pltpu.sync_copy(z_hbm.at[pl.ds(start, BLK)], z_v)
            cp_rows = pltpu.async_copy(fused_hbm.at[z_v], rows_v, sem)
            cp_ze = pltpu.async_copy(ze_hbm.at[pl.ds(start, BLK)], ze_v, sem2)
            cp_rows.wait()
            cp_ze.wait()

            def node(n, c2):
                for j in range(EMBED_DIM // 16):
                    m = ze_v[n, pl.ds(j * 16, 16)] + 1.0
                    for o in range(N_ORB):
                        cidx = o * EMBED_DIM + j * 16
                        rows_v[n, pl.ds(cidx, 16)] = (
                            rows_v[n, pl.ds(cidx, 16)] * m
                        )
                return c2

            lax.fori_loop(0, BLK, node, 0)
            pltpu.sync_copy(rows_v, out_hbm.at[pl.ds(start, BLK)])

        return carry

    lax.fori_loop(0, max_iter, blk_body, 0)


def _make_combine(n_node):
    return pl.kernel(
        functools.partial(_combine_body, n_node),
        out_type=jax.ShapeDtypeStruct((n_node, ROW), jnp.float32),
        mesh=_MESH,
        compiler_params=_SC_PARAMS,
        scratch_types=[
            pltpu.VMEM((BLK,), jnp.int32),
            pltpu.VMEM((BLK, EMBED_DIM), jnp.float32),
            pltpu.VMEM((BLK, ROW), jnp.float32),
            pltpu.SemaphoreType.DMA,
            pltpu.SemaphoreType.DMA,
        ],
    )


def kernel(z, z_embed, embeds):
    n_node = z.shape[0]
    assert n_node % BLK == 0
    z32 = z.astype(jnp.int32)
    ef = embeds.reshape(N_ORB * 5, EMBED_DIM)
    idx_const = jnp.asarray(_elec_idx_const())
    fused = _fuse(ef, idx_const)
    fused_rows = fused.reshape(MAX_Z + 1, ROW)
    out = _make_combine(n_node)(z32, z_embed, fused_rows)
    return out.reshape(n_node, N_ORB, EMBED_DIM)


# speedup vs baseline: 9.0541x; 1.4006x over previous
import functools

import jax
import jax.numpy as jnp
import numpy as np
from jax import lax
from jax.experimental import pallas as pl
from jax.experimental.pallas import tpu as pltpu
from jax.experimental.pallas import tpu_sc as plsc

MAX_Z = 36
N_ORB = 13
EMBED_DIM = 64
SUB_CAPS = [2, 2, 3, 3, 2, 3, 3, 2, 4, 3, 3, 3, 3]

NC, NS = 2, 16
NW = NC * NS
BLK = 40
ROW = N_ORB * EMBED_DIM
OROWS = BLK * ROW // 128
NCH = ROW // 16


def _elec_idx_const() -> np.ndarray:
    t = np.zeros((MAX_Z + 1, N_ORB), dtype=np.int32)
    for zz in range(1, MAX_Z + 1):
        rem = zz
        for col, cap in enumerate(SUB_CAPS):
            e = min(rem, cap)
            t[zz, col] = e
            rem -= e
            if rem == 0:
                break
    idx = (np.arange(N_ORB, dtype=np.int32)[None, :] * 5 + t).reshape(-1)
    pad = np.zeros(512, dtype=np.int32)
    pad[: idx.size] = idx
    return pad.reshape(4, 128)


_MESH = plsc.VectorSubcoreMesh(core_axis_name="c", subcore_axis_name="s")
_NROWS = (MAX_Z + 1) * N_ORB
_SC_PARAMS = pltpu.CompilerParams(use_tc_tiling_on_sc=False)


def _fuse_body(ef_hbm, idx_hbm, fused_hbm, idx_v, fused_v, sem):
    wid = lax.axis_index("s") * NC + lax.axis_index("c")

    @pl.when(wid == 0)
    def _():
        pltpu.sync_copy(idx_hbm, idx_v)
        for k in range(4):
            pltpu.async_copy(
                ef_hbm.at[idx_v.at[k]], fused_v.at[pl.ds(k * 128, 128)], sem
            ).wait()
        pltpu.sync_copy(fused_v.at[pl.ds(0, _NROWS)], fused_hbm)


_fuse = pl.kernel(
    _fuse_body,
    out_type=jax.ShapeDtypeStruct((_NROWS, EMBED_DIM), jnp.float32),
    mesh=_MESH,
    compiler_params=_SC_PARAMS,
    scratch_types=[
        pltpu.VMEM((4, 128), jnp.int32),
        pltpu.VMEM((512, EMBED_DIM), jnp.float32),
        pltpu.SemaphoreType.DMA,
    ],
)


def _combine_body(n_node, z_hbm, ze_hbm, fused_hbm, out_hbm,
                  z_v, ze_v, gbuf, obuf, sem_g, sem_z, sem_o):
    wid = lax.axis_index("s") * NC + lax.axis_index("c")
    n_blocks = n_node // BLK
    max_iter = (n_blocks + NW - 1) // NW

    def issue_in(i, p):
        blk = i * NW + wid

        @pl.when(blk < n_blocks)
        def _():
            start = blk * BLK
            pltpu.sync_copy(z_hbm.at[pl.ds(start, BLK)], z_v.at[p])
            pltpu.async_copy(fused_hbm.at[z_v.at[p]], gbuf.at[p], sem_g)
            pltpu.async_copy(ze_hbm.at[pl.ds(start, BLK)], ze_v.at[p], sem_z)

    def compute(p):
        def pair(t, carry):
            m = []
            for nn in range(2):
                for j in range(4):
                    m.append(ze_v[p, 2 * t + nn, pl.ds(j * 16, 16)] + 1.0)
            for q in range(2 * NCH):
                nn, c = q // NCH, q % NCH
                flat = nn * ROW + c * 16
                obuf[13 * t + flat // 128, pl.ds(flat % 128, 16)] = (
                    gbuf[p, 2 * t + nn, pl.ds(c * 16, 16)] * m[nn * 4 + c % 4]
                )
            return carry

        lax.fori_loop(0, BLK // 2, pair, 0)

    issue_in(0, 0)

    def blk_body(i, carry):
        p = lax.rem(i, 2)
        issue_in(i + 1, 1 - p)
        blk = i * NW + wid

        @pl.when(blk < n_blocks)
        def _():
            pltpu.make_async_copy(
                fused_hbm.at[z_v.at[p]], gbuf.at[p], sem_g).wait()
            pltpu.make_async_copy(
                ze_hbm.at[pl.ds(blk * BLK, BLK)], ze_v.at[p], sem_z).wait()

            @pl.when(i >= 1)
            def _():
                prev = (i - 1) * NW + wid
                pltpu.make_async_copy(
                    obuf, out_hbm.at[pl.ds(prev * OROWS, OROWS)], sem_o
                ).wait()

            compute(p)
            pltpu.async_copy(
                obuf, out_hbm.at[pl.ds(blk * OROWS, OROWS)], sem_o)

        return carry

    lax.fori_loop(0, max_iter, blk_body, 0)

    last = (n_blocks - 1 - wid) // NW * NW + wid
    pltpu.make_async_copy(
        obuf, out_hbm.at[pl.ds(last * OROWS, OROWS)], sem_o).wait()


def _make_combine(n_node):
    return pl.kernel(
        functools.partial(_combine_body, n_node),
        out_type=jax.ShapeDtypeStruct((n_node * ROW // 128, 128), jnp.float32),
        mesh=_MESH,
        compiler_params=_SC_PARAMS,
        scratch_types=[
            pltpu.VMEM((2, BLK), jnp.int32),
            pltpu.VMEM((2, BLK, EMBED_DIM), jnp.float32),
            pltpu.VMEM((2, BLK, ROW), jnp.float32),
            pltpu.VMEM((OROWS, 128), jnp.float32),
            pltpu.SemaphoreType.DMA,
            pltpu.SemaphoreType.DMA,
            pltpu.SemaphoreType.DMA,
        ],
    )


def kernel(z, z_embed, embeds):
    n_node = z.shape[0]
    assert n_node % BLK == 0 and (BLK * ROW) % 128 == 0
    z32 = z.astype(jnp.int32)
    ef = embeds.reshape(N_ORB * 5, EMBED_DIM)
    idx_const = jnp.asarray(_elec_idx_const())
    fused = _fuse(ef, idx_const)
    fused_rows = fused.reshape(MAX_Z + 1, ROW)
    out = _make_combine(n_node)(z32, z_embed, fused_rows)
    return out.reshape(n_node, N_ORB, EMBED_DIM)


# speedup vs baseline: 10.2537x; 1.1325x over previous
import functools

import jax
import jax.numpy as jnp
import numpy as np
from jax import lax
from jax.experimental import pallas as pl
from jax.experimental.pallas import tpu as pltpu
from jax.experimental.pallas import tpu_sc as plsc

MAX_Z = 36
N_ORB = 13
EMBED_DIM = 64
SUB_CAPS = [2, 2, 3, 3, 2, 3, 3, 2, 4, 3, 3, 3, 3]

NC, NS = 2, 16
NW = NC * NS
BLK = 40
ROW = N_ORB * EMBED_DIM
OROWS = BLK * ROW // 128
ZEROWS = BLK * EMBED_DIM // 128
NCH = ROW // 16


def _elec_idx_const() -> np.ndarray:
    t = np.zeros((MAX_Z + 1, N_ORB), dtype=np.int32)
    for zz in range(1, MAX_Z + 1):
        rem = zz
        for col, cap in enumerate(SUB_CAPS):
            e = min(rem, cap)
            t[zz, col] = e
            rem -= e
            if rem == 0:
                break
    idx = (np.arange(N_ORB, dtype=np.int32)[None, :] * 5 + t).reshape(-1)
    pad = np.zeros(512, dtype=np.int32)
    pad[: idx.size] = idx
    return pad.reshape(4, 128)


_MESH = plsc.VectorSubcoreMesh(core_axis_name="c", subcore_axis_name="s")
_NROWS = (MAX_Z + 1) * N_ORB
_SC_PARAMS = pltpu.CompilerParams(use_tc_tiling_on_sc=False)


def _fuse_body(ef_hbm, idx_hbm, fused_hbm, idx_v, fused_v, sem):
    wid = lax.axis_index("s") * NC + lax.axis_index("c")

    @pl.when(wid == 0)
    def _():
        pltpu.sync_copy(idx_hbm, idx_v)
        for k in range(4):
            pltpu.async_copy(
                ef_hbm.at[idx_v.at[k]], fused_v.at[pl.ds(k * 128, 128)], sem
            ).wait()
        pltpu.sync_copy(fused_v.at[pl.ds(0, _NROWS)], fused_hbm)


_fuse = pl.kernel(
    _fuse_body,
    out_type=jax.ShapeDtypeStruct((_NROWS, EMBED_DIM), jnp.float32),
    mesh=_MESH,
    compiler_params=_SC_PARAMS,
    scratch_types=[
        pltpu.VMEM((4, 128), jnp.int32),
        pltpu.VMEM((512, EMBED_DIM), jnp.float32),
        pltpu.SemaphoreType.DMA,
    ],
)


def _combine_body(n_node, z_hbm, ze_hbm, fused_hbm, out_hbm,
                  z_v, ze_v, gbuf, obuf, sem_g, sem_z, sem_o):
    wid = lax.axis_index("s") * NC + lax.axis_index("c")
    n_blocks = n_node // BLK
    base_n = n_blocks // NW
    n_extra = n_blocks - base_n * NW
    nblk = base_n + jnp.where(wid < n_extra, 1, 0)
    base_blk = wid * base_n + jnp.minimum(wid, n_extra)
    max_iter = base_n + (1 if n_extra else 0)

    pltpu.sync_copy(
        z_hbm.at[pl.ds(base_blk * BLK, (base_n + 1) * BLK)], z_v)

    def issue_in(i):
        @pl.when(i < nblk)
        def _():
            blk = base_blk + i
            pltpu.async_copy(
                fused_hbm.at[z_v.at[pl.ds(i * BLK, BLK)]],
                gbuf.at[lax.rem(i, 2)], sem_g)
            pltpu.async_copy(
                ze_hbm.at[pl.ds(blk * ZEROWS, ZEROWS)],
                ze_v.at[lax.rem(i, 2)], sem_z)

    def compute(p):
        @plsc.parallel_loop(0, BLK // 2, unroll=2)
        def pair(t):
            m = []
            for nn in range(2):
                for j in range(EMBED_DIM // 16):
                    m.append(
                        ze_v[p, t, pl.ds(nn * EMBED_DIM + j * 16, 16)] + 1.0)
            for q in range(2 * NCH):
                nn, c = q // NCH, q % NCH
                flat = nn * ROW + c * 16
                obuf[13 * t + flat // 128, pl.ds(flat % 128, 16)] = (
                    gbuf[p, 2 * t + nn, pl.ds(c * 16, 16)] * m[nn * 4 + c % 4]
                )

    issue_in(0)

    def blk_body(i, carry):
        p = lax.rem(i, 2)
        issue_in(i + 1)

        @pl.when(i < nblk)
        def _():
            blk = base_blk + i
            pltpu.make_async_copy(
                fused_hbm.at[z_v.at[pl.ds(i * BLK, BLK)]],
                gbuf.at[p], sem_g).wait()
            pltpu.make_async_copy(
                ze_hbm.at[pl.ds(blk * ZEROWS, ZEROWS)],
                ze_v.at[p], sem_z).wait()

            @pl.when(i >= 1)
            def _():
                pltpu.make_async_copy(
                    obuf, out_hbm.at[pl.ds((blk - 1) * OROWS, OROWS)], sem_o
                ).wait()

            compute(p)
            pltpu.async_copy(
                obuf, out_hbm.at[pl.ds(blk * OROWS, OROWS)], sem_o)

        return carry

    lax.fori_loop(0, max_iter, blk_body, 0)

    last = base_blk + nblk - 1
    pltpu.make_async_copy(
        obuf, out_hbm.at[pl.ds(last * OROWS, OROWS)], sem_o).wait()


def _make_combine(n_node):
    return pl.kernel(
        functools.partial(_combine_body, n_node),
        out_type=jax.ShapeDtypeStruct((n_node * ROW // 128, 128), jnp.float32),
        mesh=_MESH,
        compiler_params=_SC_PARAMS,
        scratch_types=[
            pltpu.VMEM(((1250 // NW + 1) * BLK,), jnp.int32),
            pltpu.VMEM((2, ZEROWS, 128), jnp.float32),
            pltpu.VMEM((2, BLK, ROW), jnp.float32),
            pltpu.VMEM((OROWS, 128), jnp.float32),
            pltpu.SemaphoreType.DMA,
            pltpu.SemaphoreType.DMA,
            pltpu.SemaphoreType.DMA,
        ],
    )


def kernel(z, z_embed, embeds):
    n_node = z.shape[0]
    assert n_node % BLK == 0 and (BLK * ROW) % 128 == 0
    n_blocks = n_node // BLK
    base_n = n_blocks // NW
    z32 = jnp.pad(z.astype(jnp.int32), (0, NW * (base_n + 1) * BLK - n_node))
    ze2 = z_embed.reshape(n_node * EMBED_DIM // 128, 128)
    ef = embeds.reshape(N_ORB * 5, EMBED_DIM)
    idx_const = jnp.asarray(_elec_idx_const())
    fused = _fuse(ef, idx_const)
    fused_rows = fused.reshape(MAX_Z + 1, ROW)
    out = _make_combine(n_node)(z32, ze2, fused_rows)
    return out.reshape(n_node, N_ORB, EMBED_DIM)
